# Initial kernel scaffold; baseline (speedup 1.0000x reference)
import jax
import jax.numpy as jnp
from jax.experimental import pallas as pl


def kernel(input_features, neighbor_inds, guidance, weightnet):
    raise NotImplementedError("write your pallas kernel here")


# speedup vs baseline: 2.4173x; 2.4173x over previous
import functools

import jax
import jax.numpy as jnp
import numpy as np
from jax import lax
from jax.experimental import pallas as pl
from jax.experimental.pallas import tpu as pltpu
from jax.experimental.pallas import tpu_sc as plsc

_N = 10000
_C = 128
_M = 10000
_K = 32
_H = 8
_CM = 16

_MK = _M * _K
_RB = 128
_NW = 32
_NBLK = _MK // _RB
_BASE_BLKS = _NBLK // _NW
_EXTRA = _NBLK % _NW

_E = np.kron(np.eye(_H, dtype=np.float32), np.ones((1, _C // _H), np.float32))

_MB = 16
_RWS = _MB * _K


@functools.partial(
    pl.kernel,
    mesh=plsc.VectorSubcoreMesh(core_axis_name="c", subcore_axis_name="s"),
    out_type=jax.ShapeDtypeStruct((_MK, _C), jnp.float32),
    scratch_types=[
        pltpu.VMEM((_RB,), jnp.int32),
        pltpu.VMEM((_RB, _C), jnp.float32),
        pltpu.SemaphoreType.DMA,
    ],
)
def _sc_gather(feat_hbm, idx_hbm, out_hbm, idx_v, rows_v, sem):
    wid = lax.axis_index("s") * 2 + lax.axis_index("c")
    nb = _BASE_BLKS + jnp.where(wid < _EXTRA, 1, 0)

    def block_body(j, carry):
        rbase = _RB * (wid + _NW * j)
        pltpu.sync_copy(idx_hbm.at[pl.ds(rbase, _RB)], idx_v)
        pltpu.async_copy(feat_hbm.at[idx_v], rows_v, sem).wait()
        pltpu.sync_copy(rows_v, out_hbm.at[pl.ds(rbase, _RB)])
        return carry

    lax.fori_loop(0, nb, block_body, 0)


def _tc_body(e_ref, g_ref, guid_ref, w_ref, o_ref):
    guid_exp = jnp.dot(guid_ref[...], e_ref[...],
                       preferred_element_type=jnp.float32)
    g = g_ref[...] * guid_exp
    w = w_ref[...]
    for p in range(_MB):
        gp = g[p * _K:(p + 1) * _K, :]
        wp = w[p * _K:(p + 1) * _K, :]
        o_ref[p] = lax.dot_general(
            gp, wp, (((0,), (0,)), ((), ())),
            preferred_element_type=jnp.float32)


_tc_einsum = pl.pallas_call(
    _tc_body,
    grid=(_M // _MB,),
    in_specs=[
        pl.BlockSpec((_H, _C), lambda i: (0, 0)),
        pl.BlockSpec((_RWS, _C), lambda i: (i, 0)),
        pl.BlockSpec((_RWS, _H), lambda i: (i, 0)),
        pl.BlockSpec((_RWS, _CM), lambda i: (i, 0)),
    ],
    out_specs=pl.BlockSpec((_MB, _C, _CM), lambda i: (i, 0, 0)),
    out_shape=jax.ShapeDtypeStruct((_M, _C, _CM), jnp.float32),
)


def kernel(input_features, neighbor_inds, guidance, weightnet):
    B, N, C = input_features.shape
    _, M, K = neighbor_inds.shape
    feat = input_features.reshape(N, C)
    idx = neighbor_inds.reshape(M * K)
    guid = guidance.reshape(M * K, _H)
    w = weightnet.reshape(M * K, _CM)
    gathered = _sc_gather(feat, idx)
    out = _tc_einsum(jnp.asarray(_E), gathered, guid, w)
    return out.reshape(B, M, C * _CM)
